# Initial kernel scaffold; baseline (speedup 1.0000x reference)
import jax
import jax.numpy as jnp
from jax.experimental import pallas as pl


def kernel(x, edge_index, edge_attr, keys_table, values_table, W1, b1, W2, b2, eps, ln_g, ln_b):
    raise NotImplementedError("write your pallas kernel here")


# speedup vs baseline: 2.9250x; 2.9250x over previous
import functools

import jax
import jax.numpy as jnp
from jax import lax
from jax.experimental import pallas as pl
from jax.experimental.pallas import tpu as pltpu
from jax.experimental.pallas import tpu_sc as plsc

N = 10000
E = 320000
D = 128
V = 1001
L = 4

NC = 2
NS = 16
NW = NC * NS
LANES = 16
C = 80

_MESH = dict(core_axis_name="c", subcore_axis_name="s", num_cores=NC,
             num_subcores=NS)


def _embed_sc(xa, xb, keys_table, values_table):
    n_chunks = N // C

    @functools.partial(
        pl.kernel,
        out_type=jax.ShapeDtypeStruct((N, D), jnp.float32),
        mesh=plsc.VectorSubcoreMesh(**_MESH),
        scratch_types=[
            pltpu.VMEM((C,), jnp.int32),
            pltpu.VMEM((C,), jnp.int32),
            pltpu.VMEM((C, D), jnp.float32),
            pltpu.VMEM((C, D), jnp.float32),
        ],
    )
    def k(xa_hbm, xb_hbm, keys_hbm, values_hbm, out_hbm, ia, ib, ka, vb):
        wid = lax.axis_index("c") * NS + lax.axis_index("s")

        @pl.loop(0, (n_chunks + NW - 1) // NW)
        def _(i):
            chunk = wid + NW * i

            @pl.when(chunk < n_chunks)
            def _():
                base = chunk * C
                pltpu.sync_copy(xa_hbm.at[pl.ds(base, C)], ia)
                pltpu.sync_copy(xb_hbm.at[pl.ds(base, C)], ib)
                pltpu.sync_copy(keys_hbm.at[ia], ka)
                pltpu.sync_copy(values_hbm.at[ib], vb)

                @pl.loop(0, C)
                def _(r):
                    for j in range(D // LANES):
                        sl = pl.ds(j * LANES, LANES)
                        ka[r, sl] = ka[r, sl] + vb[r, sl]

                pltpu.sync_copy(ka, out_hbm.at[pl.ds(base, C)])

    return k(xa, xb, keys_table, values_table)


def _edge_sc(h, src, dst, edge_attr):
    n_chunks = N // C
    e_per_w = E // NW
    e_chunks = e_per_w // C

    @functools.partial(
        pl.kernel,
        out_type=jax.ShapeDtypeStruct((NC, N, D), jnp.float32),
        mesh=plsc.VectorSubcoreMesh(**_MESH),
        scratch_types=[
            pltpu.VMEM((C,), jnp.int32),
            pltpu.VMEM((C,), jnp.int32),
            pltpu.VMEM((C, D), jnp.float32),
            pltpu.VMEM((C, D), jnp.float32),
            pltpu.VMEM_SHARED((N, D), jnp.float32),
        ],
    )
    def k(h_hbm, src_hbm, dst_hbm, ea_hbm, out_hbm, sidx, didx, hs, ea, acc):
        cid = lax.axis_index("c")
        sid = lax.axis_index("s")
        wid = cid * NS + sid

        @pl.loop(0, C)
        def _(r):
            for j in range(D // LANES):
                hs[r, pl.ds(j * LANES, LANES)] = jnp.zeros((LANES,), jnp.float32)

        @pl.loop(0, (n_chunks + NS - 1) // NS)
        def _(i):
            chunk = sid + NS * i

            @pl.when(chunk < n_chunks)
            def _():
                pltpu.sync_copy(hs, acc.at[pl.ds(chunk * C, C)])

        plsc.subcore_barrier()

        @pl.loop(0, e_chunks)
        def _(i):
            base = wid * e_per_w + i * C
            pltpu.sync_copy(src_hbm.at[pl.ds(base, C)], sidx)
            pltpu.sync_copy(dst_hbm.at[pl.ds(base, C)], didx)
            pltpu.sync_copy(h_hbm.at[sidx], hs)
            pltpu.sync_copy(ea_hbm.at[pl.ds(base, C)], ea)

            @pl.loop(0, C)
            def _(r):
                for j in range(D // LANES):
                    sl = pl.ds(j * LANES, LANES)
                    hs[r, sl] = jnp.maximum(hs[r, sl] + ea[r, sl], 0.0)

            pltpu.sync_copy(hs, acc.at[didx], add=True)

        plsc.subcore_barrier()

        @pl.loop(0, (n_chunks + NS - 1) // NS)
        def _(i):
            chunk = sid + NS * i

            @pl.when(chunk < n_chunks)
            def _():
                sl = pl.ds(chunk * C, C)
                pltpu.sync_copy(acc.at[sl], out_hbm.at[cid].at[sl])

    return k(h, src, dst, edge_attr)


def _mlp_tc(h, a0, a1, W1l, b1l, W2l, b2l, eps1, g, b):
    BLK = 400
    grid = (N // BLK,)

    def body(h_ref, a0_ref, a1_ref, w1_ref, b1_ref, w2_ref, b2_ref, e_ref,
             g_ref, bb_ref, o_ref):
        hv = h_ref[...]
        z = e_ref[0, 0] * hv + a0_ref[...] + a1_ref[...]
        u = jnp.maximum(
            jnp.dot(z, w1_ref[...], preferred_element_type=jnp.float32)
            + b1_ref[...], 0.0)
        v = jnp.dot(u, w2_ref[...], preferred_element_type=jnp.float32) + b2_ref[...]
        z2 = v + hv
        mu = jnp.mean(z2, axis=-1, keepdims=True)
        zc = z2 - mu
        var = jnp.mean(zc * zc, axis=-1, keepdims=True)
        o_ref[...] = zc * lax.rsqrt(var + 1e-5) * g_ref[...] + bb_ref[...]

    row_spec = pl.BlockSpec((BLK, D), lambda i: (i, 0))
    full = lambda shape: pl.BlockSpec(shape, lambda i: tuple(0 for _ in shape))
    return pl.pallas_call(
        body,
        grid=grid,
        in_specs=[
            row_spec, row_spec,
            pl.BlockSpec((BLK, D), lambda i: (i, 0)),
            full((D, 2 * D)), full((1, 2 * D)),
            full((2 * D, D)), full((1, D)),
            full((1, 1)), full((1, D)), full((1, D)),
        ],
        out_specs=row_spec,
        out_shape=jax.ShapeDtypeStruct((N, D), jnp.float32),
    )(h, a0, a1, W1l, b1l, W2l, b2l, eps1, g, b)


def kernel(x, edge_index, edge_attr, keys_table, values_table, W1, b1, W2, b2,
           eps, ln_g, ln_b):
    xa = x[:, 0].astype(jnp.int32)
    xb = x[:, 1].astype(jnp.int32)
    src = edge_index[0].astype(jnp.int32)
    dst = edge_index[1].astype(jnp.int32)

    h = _embed_sc(xa, xb, keys_table, values_table)
    for l in range(L):
        part = _edge_sc(h, src, dst, edge_attr)
        h = _mlp_tc(
            h, part[0], part[1],
            W1[l], b1[l].reshape(1, 2 * D),
            W2[l], b2[l].reshape(1, D),
            (1.0 + eps[l]).reshape(1, 1),
            ln_g[l].reshape(1, D), ln_b[l].reshape(1, D),
        )
    return h


# speedup vs baseline: 3.5109x; 1.2003x over previous
import functools

import jax
import jax.numpy as jnp
from jax import lax
from jax.experimental import pallas as pl
from jax.experimental.pallas import tpu as pltpu
from jax.experimental.pallas import tpu_sc as plsc

N = 10000
E = 320000
D = 128
V = 1001
L = 4

NC = 2
NS = 16
NW = NC * NS
LANES = 16
C = 80
CE = 40

_MESH = dict(core_axis_name="c", subcore_axis_name="s", num_cores=NC,
             num_subcores=NS)


def _embed_sc(xa, xb, keys_table, values_table):
    n_chunks = N // C

    @functools.partial(
        pl.kernel,
        out_type=jax.ShapeDtypeStruct((N, D), jnp.float32),
        mesh=plsc.VectorSubcoreMesh(**_MESH),
        scratch_types=[
            pltpu.VMEM((C,), jnp.int32),
            pltpu.VMEM((C,), jnp.int32),
            pltpu.VMEM((C, D), jnp.float32),
            pltpu.VMEM((C, D), jnp.float32),
        ],
    )
    def k(xa_hbm, xb_hbm, keys_hbm, values_hbm, out_hbm, ia, ib, ka, vb):
        wid = lax.axis_index("c") * NS + lax.axis_index("s")

        @pl.loop(0, (n_chunks + NW - 1) // NW)
        def _(i):
            chunk = wid + NW * i

            @pl.when(chunk < n_chunks)
            def _():
                base = chunk * C
                pltpu.sync_copy(xa_hbm.at[pl.ds(base, C)], ia)
                pltpu.sync_copy(xb_hbm.at[pl.ds(base, C)], ib)
                pltpu.sync_copy(keys_hbm.at[ia], ka)
                pltpu.sync_copy(values_hbm.at[ib], vb)

                @pl.loop(0, C)
                def _(r):
                    for j in range(D // LANES):
                        sl = pl.ds(j * LANES, LANES)
                        ka[r, sl] = ka[r, sl] + vb[r, sl]

                pltpu.sync_copy(ka, out_hbm.at[pl.ds(base, C)])

    return k(xa, xb, keys_table, values_table)


def _edge_sc(h, src3, dst3, edge_attr):
    n_chunks = N // C
    e_per_w = E // NW
    e_chunks = e_per_w // CE
    NB = 4

    @functools.partial(
        pl.kernel,
        out_type=jax.ShapeDtypeStruct((NC, N, D), jnp.float32),
        mesh=plsc.VectorSubcoreMesh(**_MESH),
        scratch_types=(
            [pltpu.VMEM((CE,), jnp.int32)] * NB
            + [pltpu.VMEM((CE,), jnp.int32)] * NB
            + [pltpu.VMEM((CE, D), jnp.float32)] * NB
            + [pltpu.VMEM((CE, D), jnp.float32)] * NB
            + [pltpu.VMEM_SHARED((N, D), jnp.float32)]
            + [pltpu.SemaphoreType.DMA] * (4 * NB)
        ),
    )
    def k(h_hbm, src_hbm, dst_hbm, ea_hbm, out_hbm, *refs):
        sidx = refs[0:NB]
        didx = refs[NB:2 * NB]
        hs = refs[2 * NB:3 * NB]
        ea = refs[3 * NB:4 * NB]
        acc = refs[4 * NB]
        isem = refs[4 * NB + 1:4 * NB + 1 + NB]
        gsem = refs[4 * NB + 1 + NB:4 * NB + 1 + 2 * NB]
        esem = refs[4 * NB + 1 + 2 * NB:4 * NB + 1 + 3 * NB]
        ssem = refs[4 * NB + 1 + 3 * NB:4 * NB + 1 + 4 * NB]

        cid = lax.axis_index("c")
        sid = lax.axis_index("s")
        wid = cid * NS + sid
        ebase = wid * e_per_w

        def fire_idx(c, b):
            pltpu.async_copy(src_hbm.at[pl.ds(pl.multiple_of(ebase + c * CE, 8), CE)], sidx[b], isem[b])
            pltpu.async_copy(dst_hbm.at[pl.ds(pl.multiple_of(ebase + c * CE, 8), CE)], didx[b], isem[b])

        def wait_idx(b):
            pltpu.make_async_copy(src_hbm.at[pl.ds(0, CE)], sidx[b], isem[b]).wait()
            pltpu.make_async_copy(dst_hbm.at[pl.ds(0, CE)], didx[b], isem[b]).wait()

        def fire_data(c, b):
            pltpu.async_copy(h_hbm.at[sidx[b]], hs[b], gsem[b])
            pltpu.async_copy(ea_hbm.at[pl.ds(pl.multiple_of(ebase + c * CE, 8), CE)], ea[b], esem[b])

        def wait_data(b):
            pltpu.make_async_copy(h_hbm.at[sidx[b]], hs[b], gsem[b]).wait()
            pltpu.make_async_copy(ea_hbm.at[pl.ds(0, CE)], ea[b], esem[b]).wait()

        def wait_scat(b):
            pltpu.make_async_copy(hs[b], acc.at[didx[b]], ssem[b]).wait()

        nz = N // CE

        @pl.loop(0, CE)
        def _(r):
            for j in range(D // LANES):
                hs[0][r, pl.ds(j * LANES, LANES)] = jnp.zeros((LANES,), jnp.float32)

        @pl.loop(0, (nz + NS - 1) // NS)
        def _(i):
            chunk = sid + NS * i

            @pl.when(chunk < nz)
            def _():
                pltpu.sync_copy(hs[0], acc.at[pl.ds(pl.multiple_of(chunk * CE, 8), CE)])

        plsc.subcore_barrier()

        fire_idx(0, 0)
        fire_idx(1, 1)
        fire_idx(2, 2)
        wait_idx(0)
        fire_data(0, 0)
        wait_idx(1)
        fire_data(1, 1)

        @pl.loop(0, e_chunks, step=NB)
        def _(i):
            for b in range(NB):
                c = i + b

                @pl.when(c < e_chunks)
                def _():
                    wait_data(b)

                    @pl.loop(0, CE, unroll=2)
                    def _(r):
                        for j in range(D // LANES):
                            sl = pl.ds(j * LANES, LANES)
                            hs[b][r, sl] = jnp.maximum(hs[b][r, sl] + ea[b][r, sl], 0.0)

                    pltpu.async_copy(hs[b], acc.at[didx[b]], ssem[b], add=True)

                    bn = (b + 3) % NB

                    @pl.when(c + 3 < e_chunks)
                    def _():
                        @pl.when(c >= 1)
                        def _():
                            wait_scat(bn)

                        fire_idx(c + 3, bn)

                    bg = (b + 2) % NB

                    @pl.when(c + 2 < e_chunks)
                    def _():
                        wait_idx(bg)
                        fire_data(c + 2, bg)

        for b in range(NB):
            wait_scat((e_chunks - NB + b) % NB)

        plsc.subcore_barrier()

        @pl.loop(0, (n_chunks + NS - 1) // NS)
        def _(i):
            chunk = sid + NS * i

            @pl.when(chunk < n_chunks)
            def _():
                sl = pl.ds(pl.multiple_of(chunk * C, 8), C)
                pltpu.sync_copy(acc.at[sl], out_hbm.at[cid].at[sl])

    return k(h, src3, dst3, edge_attr)


def _mlp_tc(h, a0, a1, W1l, b1l, W2l, b2l, eps1, g, b):
    BLK = 400
    grid = (N // BLK,)

    def body(h_ref, a0_ref, a1_ref, w1_ref, b1_ref, w2_ref, b2_ref, e_ref,
             g_ref, bb_ref, o_ref):
        hv = h_ref[...]
        z = e_ref[0, 0] * hv + a0_ref[...] + a1_ref[...]
        u = jnp.maximum(
            jnp.dot(z, w1_ref[...], preferred_element_type=jnp.float32)
            + b1_ref[...], 0.0)
        v = jnp.dot(u, w2_ref[...], preferred_element_type=jnp.float32) + b2_ref[...]
        z2 = v + hv
        mu = jnp.mean(z2, axis=-1, keepdims=True)
        zc = z2 - mu
        var = jnp.mean(zc * zc, axis=-1, keepdims=True)
        o_ref[...] = zc * lax.rsqrt(var + 1e-5) * g_ref[...] + bb_ref[...]

    row_spec = pl.BlockSpec((BLK, D), lambda i: (i, 0))
    full = lambda shape: pl.BlockSpec(shape, lambda i: tuple(0 for _ in shape))
    return pl.pallas_call(
        body,
        grid=grid,
        in_specs=[
            row_spec, row_spec,
            pl.BlockSpec((BLK, D), lambda i: (i, 0)),
            full((D, 2 * D)), full((1, 2 * D)),
            full((2 * D, D)), full((1, D)),
            full((1, 1)), full((1, D)), full((1, D)),
        ],
        out_specs=row_spec,
        out_shape=jax.ShapeDtypeStruct((N, D), jnp.float32),
    )(h, a0, a1, W1l, b1l, W2l, b2l, eps1, g, b)


def kernel(x, edge_index, edge_attr, keys_table, values_table, W1, b1, W2, b2,
           eps, ln_g, ln_b):
    xa = x[:, 0].astype(jnp.int32)
    xb = x[:, 1].astype(jnp.int32)
    src3 = edge_index[0].astype(jnp.int32)
    dst3 = edge_index[1].astype(jnp.int32)

    h = _embed_sc(xa, xb, keys_table, values_table)
    for l in range(L):
        part = _edge_sc(h, src3, dst3, edge_attr)
        h = _mlp_tc(
            h, part[0], part[1],
            W1[l], b1[l].reshape(1, 2 * D),
            W2[l], b2[l].reshape(1, D),
            (1.0 + eps[l]).reshape(1, 1),
            ln_g[l].reshape(1, D), ln_b[l].reshape(1, D),
        )
    return h


# speedup vs baseline: 6.9786x; 1.9877x over previous
import functools

import jax
import jax.numpy as jnp
from jax import lax
from jax.experimental import pallas as pl
from jax.experimental.pallas import tpu as pltpu
from jax.experimental.pallas import tpu_sc as plsc

N = 10000
E = 320000
D = 128
V = 1001
L = 4

NC = 2
NS = 16
NW = NC * NS
LANES = 16
C = 80
CE = 40

_MESH = dict(core_axis_name="c", subcore_axis_name="s", num_cores=NC,
             num_subcores=NS)


def _embed_sc(xa, xb, keys_table, values_table):
    n_chunks = N // C

    @functools.partial(
        pl.kernel,
        out_type=jax.ShapeDtypeStruct((N, D), jnp.float32),
        mesh=plsc.VectorSubcoreMesh(**_MESH),
        scratch_types=[
            pltpu.VMEM((C,), jnp.int32),
            pltpu.VMEM((C,), jnp.int32),
            pltpu.VMEM((C, D), jnp.float32),
            pltpu.VMEM((C, D), jnp.float32),
        ],
    )
    def k(xa_hbm, xb_hbm, keys_hbm, values_hbm, out_hbm, ia, ib, ka, vb):
        wid = lax.axis_index("c") * NS + lax.axis_index("s")

        @pl.loop(0, (n_chunks + NW - 1) // NW)
        def _(i):
            chunk = wid + NW * i

            @pl.when(chunk < n_chunks)
            def _():
                base = chunk * C
                pltpu.sync_copy(xa_hbm.at[pl.ds(base, C)], ia)
                pltpu.sync_copy(xb_hbm.at[pl.ds(base, C)], ib)
                pltpu.sync_copy(keys_hbm.at[ia], ka)
                pltpu.sync_copy(values_hbm.at[ib], vb)

                @plsc.parallel_loop(0, C, unroll=4)
                def _(r):
                    for j in range(D // LANES):
                        sl = pl.ds(j * LANES, LANES)
                        ka[r, sl] = ka[r, sl] + vb[r, sl]

                pltpu.sync_copy(ka, out_hbm.at[pl.ds(base, C)])

    return k(xa, xb, keys_table, values_table)


def _edge_sc(h, src3, dst3, edge_attr):
    n_chunks = N // C
    e_per_w = E // NW
    e_chunks = e_per_w // CE
    NB = 4

    @functools.partial(
        pl.kernel,
        out_type=jax.ShapeDtypeStruct((NC, N, D), jnp.float32),
        mesh=plsc.VectorSubcoreMesh(**_MESH),
        scratch_types=(
            [pltpu.VMEM((CE,), jnp.int32)] * NB
            + [pltpu.VMEM((CE,), jnp.int32)] * NB
            + [pltpu.VMEM((CE, D), jnp.float32)] * NB
            + [pltpu.VMEM((CE, D), jnp.float32)] * NB
            + [pltpu.VMEM_SHARED((N, D), jnp.float32)]
            + [pltpu.SemaphoreType.DMA] * (4 * NB)
        ),
    )
    def k(h_hbm, src_hbm, dst_hbm, ea_hbm, out_hbm, *refs):
        sidx = refs[0:NB]
        didx = refs[NB:2 * NB]
        hs = refs[2 * NB:3 * NB]
        ea = refs[3 * NB:4 * NB]
        acc = refs[4 * NB]
        isem = refs[4 * NB + 1:4 * NB + 1 + NB]
        gsem = refs[4 * NB + 1 + NB:4 * NB + 1 + 2 * NB]
        esem = refs[4 * NB + 1 + 2 * NB:4 * NB + 1 + 3 * NB]
        ssem = refs[4 * NB + 1 + 3 * NB:4 * NB + 1 + 4 * NB]

        cid = lax.axis_index("c")
        sid = lax.axis_index("s")
        wid = cid * NS + sid
        ebase = wid * e_per_w

        def fire_idx(c, b):
            pltpu.async_copy(src_hbm.at[pl.ds(pl.multiple_of(ebase + c * CE, 8), CE)], sidx[b], isem[b])
            pltpu.async_copy(dst_hbm.at[pl.ds(pl.multiple_of(ebase + c * CE, 8), CE)], didx[b], isem[b])

        def wait_idx(b):
            pltpu.make_async_copy(src_hbm.at[pl.ds(0, CE)], sidx[b], isem[b]).wait()
            pltpu.make_async_copy(dst_hbm.at[pl.ds(0, CE)], didx[b], isem[b]).wait()

        def fire_data(c, b):
            pltpu.async_copy(h_hbm.at[sidx[b]], hs[b], gsem[b])
            pltpu.async_copy(ea_hbm.at[pl.ds(pl.multiple_of(ebase + c * CE, 8), CE)], ea[b], esem[b])

        def wait_data(b):
            pltpu.make_async_copy(h_hbm.at[sidx[b]], hs[b], gsem[b]).wait()
            pltpu.make_async_copy(ea_hbm.at[pl.ds(0, CE)], ea[b], esem[b]).wait()

        def wait_scat(b):
            pltpu.make_async_copy(hs[b], acc.at[didx[b]], ssem[b]).wait()

        nz = N // CE

        @pl.loop(0, CE)
        def _(r):
            for j in range(D // LANES):
                hs[0][r, pl.ds(j * LANES, LANES)] = jnp.zeros((LANES,), jnp.float32)

        @pl.loop(0, (nz + NS - 1) // NS)
        def _(i):
            chunk = sid + NS * i

            @pl.when(chunk < nz)
            def _():
                pltpu.sync_copy(hs[0], acc.at[pl.ds(pl.multiple_of(chunk * CE, 8), CE)])

        plsc.subcore_barrier()

        fire_idx(0, 0)
        fire_idx(1, 1)
        fire_idx(2, 2)
        wait_idx(0)
        fire_data(0, 0)
        wait_idx(1)
        fire_data(1, 1)

        @pl.loop(0, e_chunks, step=NB)
        def _(i):
            for b in range(NB):
                c = i + b

                @pl.when(c < e_chunks)
                def _():
                    wait_data(b)

                    @plsc.parallel_loop(0, CE, unroll=4)
                    def _(r):
                        for j in range(D // LANES):
                            sl = pl.ds(j * LANES, LANES)
                            hs[b][r, sl] = jnp.maximum(hs[b][r, sl] + ea[b][r, sl], 0.0)

                    pltpu.async_copy(hs[b], acc.at[didx[b]], ssem[b], add=True)

                    bn = (b + 3) % NB

                    @pl.when(c + 3 < e_chunks)
                    def _():
                        @pl.when(c >= 1)
                        def _():
                            wait_scat(bn)

                        fire_idx(c + 3, bn)

                    bg = (b + 2) % NB

                    @pl.when(c + 2 < e_chunks)
                    def _():
                        wait_idx(bg)
                        fire_data(c + 2, bg)

        for b in range(NB):
            wait_scat((e_chunks - NB + b) % NB)

        plsc.subcore_barrier()

        @pl.loop(0, (n_chunks + NS - 1) // NS)
        def _(i):
            chunk = sid + NS * i

            @pl.when(chunk < n_chunks)
            def _():
                sl = pl.ds(pl.multiple_of(chunk * C, 8), C)
                pltpu.sync_copy(acc.at[sl], out_hbm.at[cid].at[sl])

    return k(h, src3, dst3, edge_attr)


def _mlp_tc(h, a0, a1, W1l, b1l, W2l, b2l, eps1, g, b):
    BLK = 400
    grid = (N // BLK,)

    def body(h_ref, a0_ref, a1_ref, w1_ref, b1_ref, w2_ref, b2_ref, e_ref,
             g_ref, bb_ref, o_ref):
        hv = h_ref[...]
        z = e_ref[0, 0] * hv + a0_ref[...] + a1_ref[...]
        u = jnp.maximum(
            jnp.dot(z, w1_ref[...], preferred_element_type=jnp.float32)
            + b1_ref[...], 0.0)
        v = jnp.dot(u, w2_ref[...], preferred_element_type=jnp.float32) + b2_ref[...]
        z2 = v + hv
        mu = jnp.mean(z2, axis=-1, keepdims=True)
        zc = z2 - mu
        var = jnp.mean(zc * zc, axis=-1, keepdims=True)
        o_ref[...] = zc * lax.rsqrt(var + 1e-5) * g_ref[...] + bb_ref[...]

    row_spec = pl.BlockSpec((BLK, D), lambda i: (i, 0))
    full = lambda shape: pl.BlockSpec(shape, lambda i: tuple(0 for _ in shape))
    return pl.pallas_call(
        body,
        grid=grid,
        in_specs=[
            row_spec, row_spec,
            pl.BlockSpec((BLK, D), lambda i: (i, 0)),
            full((D, 2 * D)), full((1, 2 * D)),
            full((2 * D, D)), full((1, D)),
            full((1, 1)), full((1, D)), full((1, D)),
        ],
        out_specs=row_spec,
        out_shape=jax.ShapeDtypeStruct((N, D), jnp.float32),
    )(h, a0, a1, W1l, b1l, W2l, b2l, eps1, g, b)


def kernel(x, edge_index, edge_attr, keys_table, values_table, W1, b1, W2, b2,
           eps, ln_g, ln_b):
    xa = x[:, 0].astype(jnp.int32)
    xb = x[:, 1].astype(jnp.int32)
    src3 = edge_index[0].astype(jnp.int32)
    dst3 = edge_index[1].astype(jnp.int32)

    h = _embed_sc(xa, xb, keys_table, values_table)
    for l in range(L):
        part = _edge_sc(h, src3, dst3, edge_attr)
        h = _mlp_tc(
            h, part[0], part[1],
            W1[l], b1[l].reshape(1, 2 * D),
            W2[l], b2[l].reshape(1, D),
            (1.0 + eps[l]).reshape(1, 1),
            ln_g[l].reshape(1, D), ln_b[l].reshape(1, D),
        )
    return h
